# Initial kernel scaffold; baseline (speedup 1.0000x reference)
import jax
import jax.numpy as jnp
from jax.experimental import pallas as pl


def kernel(x, node_norm, edge_norm, W, b, edge_index):
    raise NotImplementedError("write your pallas kernel here")


# speedup vs baseline: 16.3429x; 16.3429x over previous
import functools

import jax
import jax.numpy as jnp
from jax import lax
from jax.experimental import pallas as pl
from jax.experimental.pallas import tpu as pltpu
from jax.experimental.pallas import tpu_sc as plsc

N = 10000
E = 320000
D = 128
NEG_SLOPE = 0.01

NC = 2
NS = 16
NW = NC * NS
EPW = E // NW
C = 80
NCH = EPW // C
GRP = 5
RPS = 624
RTAIL = N - NS * RPS
LG = D // 16


def _sc_kernel():
    mesh = plsc.VectorSubcoreMesh(core_axis_name="c", subcore_axis_name="s")

    @functools.partial(
        pl.kernel,
        out_type=jax.ShapeDtypeStruct((NC, N, D), jnp.float32),
        mesh=mesh,
        scratch_types=[
            pltpu.VMEM((GRP, C), jnp.int32),
            pltpu.VMEM((GRP, C), jnp.int32),
            pltpu.VMEM((GRP, C), jnp.float32),
            pltpu.VMEM((C,), jnp.float32),
            pltpu.VMEM((C,), jnp.float32),
            pltpu.VMEM((C, D), jnp.float32),
            pltpu.VMEM_SHARED((N, D), jnp.float32),
            pltpu.SemaphoreType.DMA,
            pltpu.SemaphoreType.DMA,
        ],
        compiler_params=pltpu.CompilerParams(needs_layout_passes=False),
    )
    def k(x_hbm, nn_hbm, src_hbm, dst_hbm, en_hbm, out_hbm,
          src_v, dst_v, en_v, nns_v, scale_v, rows_v, h_shared, gsem, nsem):
        cid = lax.axis_index("c")
        sid = lax.axis_index("s")
        wid = cid * NS + sid

        zrow = jnp.zeros((16,), jnp.float32)

        def zero_body(i, _):
            rows_v[i // LG, pl.ds((i % LG) * 16, 16)] = zrow
            return 0

        lax.fori_loop(0, C * LG, zero_body, 0)
        for t in range(RPS // C):
            pltpu.sync_copy(rows_v, h_shared.at[pl.ds(sid * RPS + t * C, C)])
        rem = RPS - (RPS // C) * C
        pltpu.sync_copy(rows_v.at[pl.ds(0, rem)],
                        h_shared.at[pl.ds(sid * RPS + (RPS // C) * C, rem)])

        @pl.when(sid == NS - 1)
        def _zero_tail():
            pltpu.sync_copy(rows_v.at[pl.ds(0, RTAIL)],
                            h_shared.at[pl.ds(NS * RPS, RTAIL)])

        plsc.subcore_barrier()

        def group_body(t, _):
            pltpu.sync_copy(src_hbm.at[wid, t], src_v)
            pltpu.sync_copy(dst_hbm.at[wid, t], dst_v)
            pltpu.sync_copy(en_hbm.at[wid, t], en_v)

            def chunk_body(g, _):
                rcp = pltpu.async_copy(x_hbm.at[src_v.at[g]], rows_v, gsem)
                ncp = pltpu.async_copy(nn_hbm.at[src_v.at[g]], nns_v, nsem)
                ncp.wait()

                def scale_body(i, _):
                    sl = pl.ds(i * 16, 16)
                    scale_v[sl] = nns_v[sl] * en_v[g, sl]
                    return 0

                lax.fori_loop(0, C // 16, scale_body, 0)
                rcp.wait()

                def edge_body(e, _):
                    b = plsc.load_gather(scale_v, [jnp.full((16,), e, jnp.int32)])
                    for j in range(LG):
                        rows_v[e, pl.ds(j * 16, 16)] = rows_v[e, pl.ds(j * 16, 16)] * b
                    return 0

                lax.fori_loop(0, C, edge_body, 0)
                pltpu.sync_copy(rows_v, h_shared.at[dst_v.at[g]], add=True)
                return 0

            lax.fori_loop(0, GRP, chunk_body, 0)
            return 0

        lax.fori_loop(0, NCH // GRP, group_body, 0)

        plsc.subcore_barrier()

        pltpu.sync_copy(h_shared.at[pl.ds(sid * RPS, RPS)],
                        out_hbm.at[cid, pl.ds(sid * RPS, RPS)])

        @pl.when(sid == NS - 1)
        def _write_tail():
            pltpu.sync_copy(h_shared.at[pl.ds(NS * RPS, RTAIL)],
                            out_hbm.at[cid, pl.ds(NS * RPS, RTAIL)])

    return k


_sc_run = _sc_kernel()


def _tc_finish_body(x_ref, p_ref, nn_ref, w_ref, b_ref, o_ref):
    xmax = jnp.max(x_ref[...], axis=0, keepdims=True)
    s = (p_ref[0] + p_ref[1]) * nn_ref[...]
    hmax = jnp.max(s, axis=0, keepdims=True)
    r = 0.5 * (xmax + hmax)
    out = lax.dot_general(r, w_ref[...], (((1,), (1,)), ((), ())),
                          preferred_element_type=jnp.float32) + b_ref[...]
    o_ref[...] = jnp.where(out > 0, out, NEG_SLOPE * out)


_tc_finish = pl.pallas_call(
    _tc_finish_body,
    out_shape=jax.ShapeDtypeStruct((1, D), jnp.float32),
)


def kernel(x, node_norm, edge_norm, W, b, edge_index):
    src = edge_index[0].reshape(NW, NCH // GRP, GRP, C)
    dst = edge_index[1].reshape(NW, NCH // GRP, GRP, C)
    en = edge_norm.reshape(NW, NCH // GRP, GRP, C)
    partials = _sc_run(x, node_norm, src, dst, en)
    return _tc_finish(x, partials, node_norm.reshape(N, 1), W, b.reshape(1, D))
